# baseline (speedup 1.0000x reference)
import functools

import jax
import jax.numpy as jnp
import numpy as np
from jax import lax
from jax.experimental import pallas as pl
from jax.experimental.pallas import tpu as pltpu
from jax.experimental.pallas import tpu_sc as plsc

_B, _N, _DF = 4, 8192, 29
_GN, _GK = 1024, 32
_R2 = np.float32(0.15 * 0.15)
_DIN, _DHID, _DOUT = 32, 64, 128
_M = _B * _GN * _GK
_INV_M = 1.0 / _M
_EPS = 1e-5

_CB = 128
_GB = 128


_RW = 2 * _B
_NH = _N // 2
_CH = _GN // 2


def _pair_combine(v, op):
    up = jnp.concatenate([v[1:], v[:1]], axis=0)
    dn = jnp.concatenate([v[-1:], v[:-1]], axis=0)
    even = (lax.broadcasted_iota(jnp.int32, (_RW, 1), 0) % 2) == 0
    mate = jnp.where(even, up, dn)
    return op(v, mate)


def _fps_kernel(px_ref, py_ref, pz_ref, cx_ref, cy_ref, cz_ref):
    px = px_ref[...]
    py = py_ref[...]
    pz = pz_ref[...]
    iota_n = lax.broadcasted_iota(jnp.int32, (_RW, _NH), 1).astype(jnp.float32)
    iota_c = lax.broadcasted_iota(jnp.int32, (_RW, _CH), 1).astype(jnp.float32)
    rowhalf = (lax.broadcasted_iota(jnp.int32, (_RW, 1), 0) % 2
               ).astype(jnp.float32)
    iota_g = iota_n + float(_NH) * rowhalf

    def coords_of(last):
        lhalf = jnp.floor(last * (1.0 / _NH))
        lmod = last - lhalf * _NH
        onehot = jnp.logical_and(iota_n == lmod, rowhalf == lhalf)
        lx = _pair_combine(
            jnp.sum(jnp.where(onehot, px, 0.0), axis=1, keepdims=True),
            jnp.add)
        ly = _pair_combine(
            jnp.sum(jnp.where(onehot, py, 0.0), axis=1, keepdims=True),
            jnp.add)
        lz = _pair_combine(
            jnp.sum(jnp.where(onehot, pz, 0.0), axis=1, keepdims=True),
            jnp.add)
        return lx, ly, lz

    def write_col(col_i, val, acc):
        chalf = jnp.floor(col_i * (1.0 / _CH))
        cmod = col_i - chalf * _CH
        colm = jnp.logical_and(iota_c == cmod, rowhalf == chalf)
        return jnp.where(colm, val, acc)

    def step(i, carry):
        dists, last, cx, cy, cz = carry
        lx, ly, lz = coords_of(last)
        prev = (i - 1).astype(jnp.float32)
        cx = write_col(prev, lx, cx)
        cy = write_col(prev, ly, cy)
        cz = write_col(prev, lz, cz)
        d = (px - lx) ** 2 + (py - ly) ** 2 + (pz - lz) ** 2
        dists = jnp.minimum(dists, d)
        m = _pair_combine(jnp.max(dists, axis=1, keepdims=True),
                          jnp.maximum)
        nxt = _pair_combine(
            jnp.min(jnp.where(dists == m, iota_g, float(_N)), axis=1,
                    keepdims=True), jnp.minimum)
        return dists, nxt, cx, cy, cz

    dists0 = jnp.full((_RW, _NH), jnp.inf, jnp.float32)
    last0 = jnp.zeros((_RW, 1), jnp.float32)
    zc = jnp.zeros((_RW, _CH), jnp.float32)
    _, last, cx, cy, cz = lax.fori_loop(1, _GN, step,
                                        (dists0, last0, zc, zc, zc))
    lx, ly, lz = coords_of(last)
    fin = jnp.float32(_GN - 1)
    cx_ref[...] = write_col(fin, lx, cx)
    cy_ref[...] = write_col(fin, ly, cy)
    cz_ref[...] = write_col(fin, lz, cz)


def _fps(px, py, pz):
    shp = jax.ShapeDtypeStruct((_RW, _CH), jnp.float32)
    full = pl.BlockSpec((_RW, _NH), lambda: (0, 0))
    out = pl.BlockSpec((_RW, _CH), lambda: (0, 0))
    cx, cy, cz = pl.pallas_call(
        _fps_kernel,
        grid=(),
        in_specs=[full, full, full],
        out_specs=[out, out, out],
        out_shape=[shp, shp, shp],
    )(px.reshape(_RW, _NH), py.reshape(_RW, _NH), pz.reshape(_RW, _NH))
    return cx.reshape(_B, _GN), cy.reshape(_B, _GN), cz.reshape(_B, _GN)


def _ball_query(pxf, pyf, pzf, cxf, cyf, czf):
    n_workers = 32
    cpw = (_B * _GN) // n_workers
    n_chunk = _N // 16
    mesh = plsc.VectorSubcoreMesh(core_axis_name="c", subcore_axis_name="s")

    @functools.partial(
        pl.kernel,
        mesh=mesh,
        out_type=jax.ShapeDtypeStruct((_B * _GN * _GK,), jnp.int32),
        scratch_types=[
            pltpu.VMEM((_N,), jnp.float32),
            pltpu.VMEM((_N,), jnp.float32),
            pltpu.VMEM((_N,), jnp.float32),
            pltpu.VMEM((cpw,), jnp.float32),
            pltpu.VMEM((cpw,), jnp.float32),
            pltpu.VMEM((cpw,), jnp.float32),
            pltpu.VMEM((_GK + 64,), jnp.int32),
            pltpu.VMEM((cpw * _GK,), jnp.int32),
            pltpu.SemaphoreType.DMA,
        ],
        compiler_params=pltpu.CompilerParams(use_tc_tiling_on_sc=False,
                                             needs_layout_passes=False),
    )
    def k(px_hbm, py_hbm, pz_hbm, cx_hbm, cy_hbm, cz_hbm, out_hbm,
          px_v, py_v, pz_v, cx_v, cy_v, cz_v, row_v, out_v, sem):
        wid = lax.axis_index("s") * 2 + lax.axis_index("c")
        b = wid // (n_workers // _B)
        pltpu.sync_copy(px_hbm.at[pl.ds(b * _N, _N)], px_v)
        pltpu.sync_copy(py_hbm.at[pl.ds(b * _N, _N)], py_v)
        pltpu.sync_copy(pz_hbm.at[pl.ds(b * _N, _N)], pz_v)
        pltpu.sync_copy(cx_hbm.at[pl.ds(wid * cpw, cpw)], cx_v)
        pltpu.sync_copy(cy_hbm.at[pl.ds(wid * cpw, cpw)], cy_v)
        pltpu.sync_copy(cz_hbm.at[pl.ds(wid * cpw, cpw)], cz_v)
        lane = lax.broadcasted_iota(jnp.int32, (16,), 0)
        base_j = b * _N

        def per_center(s, carry):
            sidx = jnp.full((16,), s, jnp.int32)
            cxs = plsc.load_gather(cx_v, [sidx])
            cys = plsc.load_gather(cy_v, [sidx])
            czs = plsc.load_gather(cz_v, [sidx])

            def cond(c):
                i, cnt = c
                return jnp.logical_and(i < n_chunk // 4, cnt < _GK)

            def body(c):
                i, cnt = c
                for u in range(4):
                    off = i * 64 + u * 16
                    dx = px_v[pl.ds(off, 16)] - cxs
                    dy = py_v[pl.ds(off, 16)] - cys
                    dz = pz_v[pl.ds(off, 16)] - czs
                    d2 = dx * dx + dy * dy + dz * dz
                    m = d2 <= _R2
                    jv = lane + (off + base_j)
                    plsc.store_compressed(row_v.at[pl.ds(cnt, 16)], jv,
                                          mask=m)
                    cnt = cnt + jnp.max(plsc.all_reduce_population_count(m))
                return i + 1, cnt

            _, cnt = lax.while_loop(
                cond, body, (jnp.int32(0), jnp.int32(0)))
            csplat = jnp.full((16,), jnp.minimum(cnt, _GK), jnp.int32)
            v0 = row_v[pl.ds(0, 16)]
            fs = jnp.min(jnp.where(lane < csplat, v0, jnp.int32(2 ** 30)))
            first = jnp.full((16,), fs, jnp.int32)
            for h in range(_GK // 16):
                pos = lane + h * 16
                vh = row_v[pl.ds(h * 16, 16)]
                out_v[pl.ds(s * _GK + h * 16, 16)] = jnp.where(
                    pos < csplat, vh, first)
            return carry

        lax.fori_loop(0, cpw, per_center, 0)
        pltpu.sync_copy(out_v, out_hbm.at[pl.ds(wid * cpw * _GK, cpw * _GK)])

    return k(pxf, pyf, pzf, cxf, cyf, czf)


def _sc_gather(table, idx):
    rows = idx.shape[0]
    n_workers = 32
    per_w = rows // n_workers
    chunk = 512
    n_chunks = per_w // chunk
    mesh = plsc.VectorSubcoreMesh(core_axis_name="c", subcore_axis_name="s")

    @functools.partial(
        pl.kernel,
        mesh=mesh,
        out_type=jax.ShapeDtypeStruct((rows, _DIN), jnp.float32),
        scratch_types=[
            pltpu.VMEM((chunk,), jnp.int32),
            pltpu.VMEM((chunk, _DIN), jnp.float32),
            pltpu.SemaphoreType.DMA,
        ],
        compiler_params=pltpu.CompilerParams(use_tc_tiling_on_sc=False),
    )
    def k(table_hbm, idx_hbm, out_hbm, idx_v, rows_v, sem):
        wid = lax.axis_index("s") * 2 + lax.axis_index("c")
        base = wid * per_w

        def body(c, carry):
            start = base + c * chunk
            pltpu.sync_copy(idx_hbm.at[pl.ds(start, chunk)], idx_v)
            pltpu.async_copy(table_hbm.at[idx_v], rows_v, sem).wait()
            pltpu.sync_copy(rows_v, out_hbm.at[pl.ds(start, chunk)])
            return carry

        lax.fori_loop(0, n_chunks, body, 0)

    return k(table, idx)


def _acc_stats(st_ref, z):
    s = jnp.sum(z, axis=0, keepdims=True)
    q = jnp.sum(z * z, axis=0, keepdims=True)
    st = jnp.concatenate([s, q], axis=0)

    @pl.when(pl.program_id(0) == 0)
    def _():
        st_ref[...] = st

    @pl.when(pl.program_id(0) != 0)
    def _():
        st_ref[...] += st


def _affine(st, g, b):
    mu = st[0:1, :] * _INV_M
    var = st[1:2, :] * _INV_M - mu * mu
    inv = lax.rsqrt(var + _EPS)
    scale = g * inv
    shift = b - mu * scale
    return scale, shift


def _p1_kernel(g_ref, cx_ref, cy_ref, cz_ref, w0_ref, z0_ref, st_ref):
    g = g_ref[...]
    lane = lax.broadcasted_iota(jnp.int32, (_GB, _DIN), 1)
    sub = jnp.where(lane == _DF, cx_ref[...],
                    jnp.where(lane == _DF + 1, cy_ref[...],
                              jnp.where(lane == _DF + 2, cz_ref[...], 0.0)))
    x = g - sub[:, None, :]
    x2 = x.reshape(_GB * _GK, _DIN)
    z0 = jnp.dot(x2, w0_ref[...], preferred_element_type=jnp.float32)
    z0_ref[...] = z0.reshape(_GB, _GK, _DHID)
    _acc_stats(st_ref, z0)


def _p2_kernel(z0_ref, st0_ref, g0_ref, b0_ref, w1a_ref, w2_ref,
               z1a_ref, z2_ref, st1a_ref, st2_ref):
    scale, shift = _affine(st0_ref[...], g0_ref[...], b0_ref[...])
    z0 = z0_ref[...]
    h = jnp.maximum(z0 * scale[None] + shift[None], 0.0)
    h2 = h.reshape(_GB * _GK, _DHID)
    z1a = jnp.dot(h2, w1a_ref[...], preferred_element_type=jnp.float32)
    z2 = jnp.dot(h2, w2_ref[...], preferred_element_type=jnp.float32)
    z1a_ref[...] = z1a.reshape(_GB, _GK, _DOUT)
    z2_ref[...] = z2.reshape(_GB, _GK, _DOUT)
    _acc_stats(st1a_ref, z1a)
    _acc_stats(st2_ref, z2)


def _p3_kernel(z1a_ref, st1a_ref, g1a_ref, b1a_ref, w1b_ref,
               z1b_ref, st1b_ref):
    scale, shift = _affine(st1a_ref[...], g1a_ref[...], b1a_ref[...])
    z1a = z1a_ref[...]
    t = jnp.maximum(z1a * scale[None] + shift[None], 0.0)
    t2 = t.reshape(_GB * _GK, _DOUT)
    z1b = jnp.dot(t2, w1b_ref[...], preferred_element_type=jnp.float32)
    z1b_ref[...] = z1b.reshape(_GB, _GK, _DOUT)
    _acc_stats(st1b_ref, z1b)


def _p4_kernel(z1b_ref, z2_ref, st1b_ref, st2_ref,
               g1b_ref, b1b_ref, g2_ref, b2_ref, out_ref):
    s1b, t1b = _affine(st1b_ref[...], g1b_ref[...], b1b_ref[...])
    s2, t2 = _affine(st2_ref[...], g2_ref[...], b2_ref[...])
    a = z1b_ref[...] * s1b[None] + t1b[None]
    c = z2_ref[...] * s2[None] + t2[None]
    y = jnp.maximum(a + c, 0.0)
    out_ref[...] = jnp.max(y, axis=1)


def _mlp(gathered, cxf, cyf, czf, W0, g0, b0, W1a, g1a, b1a,
         W1b, g1b, b1b, W2, g2, b2):
    n_groups = _B * _GN
    grid = (n_groups // _GB,)
    arb = pltpu.CompilerParams(dimension_semantics=("arbitrary",))

    def blk(shape_tail):
        return pl.BlockSpec((_GB,) + shape_tail, lambda i: (i,) + (0,) * len(shape_tail))

    def full2(s):
        return pl.BlockSpec(s, lambda i: (0, 0))

    st_hid = jax.ShapeDtypeStruct((2, _DHID), jnp.float32)
    st_out = jax.ShapeDtypeStruct((2, _DOUT), jnp.float32)

    z0, st0 = pl.pallas_call(
        _p1_kernel,
        grid=grid,
        in_specs=[blk((_GK, _DIN)), blk((1,)), blk((1,)), blk((1,)),
                  full2((_DIN, _DHID))],
        out_specs=[blk((_GK, _DHID)), full2((2, _DHID))],
        out_shape=[jax.ShapeDtypeStruct((n_groups, _GK, _DHID), jnp.float32),
                   st_hid],
        compiler_params=arb,
    )(gathered, cxf, cyf, czf, W0)

    z1a, z2, st1a, st2 = pl.pallas_call(
        _p2_kernel,
        grid=grid,
        in_specs=[blk((_GK, _DHID)), full2((2, _DHID)),
                  full2((1, _DHID)), full2((1, _DHID)),
                  full2((_DHID, _DOUT)), full2((_DHID, _DOUT))],
        out_specs=[blk((_GK, _DOUT)), blk((_GK, _DOUT)),
                   full2((2, _DOUT)), full2((2, _DOUT))],
        out_shape=[jax.ShapeDtypeStruct((n_groups, _GK, _DOUT), jnp.float32),
                   jax.ShapeDtypeStruct((n_groups, _GK, _DOUT), jnp.float32),
                   st_out, st_out],
        compiler_params=arb,
    )(z0, st0, g0.reshape(1, _DHID), b0.reshape(1, _DHID), W1a, W2)

    z1b, st1b = pl.pallas_call(
        _p3_kernel,
        grid=grid,
        in_specs=[blk((_GK, _DOUT)), full2((2, _DOUT)),
                  full2((1, _DOUT)), full2((1, _DOUT)),
                  full2((_DOUT, _DOUT))],
        out_specs=[blk((_GK, _DOUT)), full2((2, _DOUT))],
        out_shape=[jax.ShapeDtypeStruct((n_groups, _GK, _DOUT), jnp.float32),
                   st_out],
        compiler_params=arb,
    )(z1a, st1a, g1a.reshape(1, _DOUT), b1a.reshape(1, _DOUT), W1b)

    f_ce = pl.pallas_call(
        _p4_kernel,
        grid=grid,
        in_specs=[blk((_GK, _DOUT)), blk((_GK, _DOUT)),
                  full2((2, _DOUT)), full2((2, _DOUT)),
                  full2((1, _DOUT)), full2((1, _DOUT)),
                  full2((1, _DOUT)), full2((1, _DOUT))],
        out_specs=blk((_DOUT,)),
        out_shape=jax.ShapeDtypeStruct((n_groups, _DOUT), jnp.float32),
        compiler_params=arb,
    )(z1b, z2, st1b, st2,
      g1b.reshape(1, _DOUT), b1b.reshape(1, _DOUT),
      g2.reshape(1, _DOUT), b2.reshape(1, _DOUT))

    return f_ce


def kernel(f, p, W0, g0, b0, W1a, g1a, b1a, W1b, g1b, b1b, W2, g2, b2):
    px = p[:, :, 0]
    py = p[:, :, 1]
    pz = p[:, :, 2]

    cx, cy, cz = _fps(px, py, pz)
    p_ce = jnp.stack([cx, cy, cz], axis=-1)

    cxf = cx.reshape(_B * _GN, 1)
    cyf = cy.reshape(_B * _GN, 1)
    czf = cz.reshape(_B * _GN, 1)
    gidx = _ball_query(px.reshape(-1), py.reshape(-1), pz.reshape(-1),
                       cx.reshape(-1), cy.reshape(-1), cz.reshape(-1))

    table = jnp.concatenate([f, p], axis=-1).reshape(_B * _N, _DIN)
    gathered = _sc_gather(table, gidx.reshape(-1))
    gathered = gathered.reshape(_B * _GN, _GK, _DIN)

    f_ce = _mlp(gathered, cxf, cyf, czf, W0, g0, b0,
                W1a, g1a, b1a, W1b, g1b, b1b, W2, g2, b2)
    return f_ce.reshape(_B, _GN, _DOUT), p_ce


# speedup vs baseline: 1.0502x; 1.0502x over previous
import functools

import jax
import jax.numpy as jnp
import numpy as np
from jax import lax
from jax.experimental import pallas as pl
from jax.experimental.pallas import tpu as pltpu
from jax.experimental.pallas import tpu_sc as plsc

_B, _N, _DF = 4, 8192, 29
_GN, _GK = 1024, 32
_R2 = np.float32(0.15 * 0.15)
_DIN, _DHID, _DOUT = 32, 64, 128
_M = _B * _GN * _GK
_INV_M = 1.0 / _M
_EPS = 1e-5

_CB = 128
_GB = 128


def _fps_kernel(px_ref, py_ref, pz_ref, cx_ref, cy_ref, cz_ref):
    px = px_ref[...]
    py = py_ref[...]
    pz = pz_ref[...]
    iota_n = lax.broadcasted_iota(jnp.int32, (_B, _N), 1).astype(jnp.float32)
    iota_c = lax.broadcasted_iota(jnp.int32, (_B, _GN), 1).astype(jnp.float32)

    def coords_of(last):
        onehot = iota_n == last
        lx = jnp.sum(jnp.where(onehot, px, 0.0), axis=1, keepdims=True)
        ly = jnp.sum(jnp.where(onehot, py, 0.0), axis=1, keepdims=True)
        lz = jnp.sum(jnp.where(onehot, pz, 0.0), axis=1, keepdims=True)
        return lx, ly, lz

    def step(i, carry):
        dists, last, cx, cy, cz = carry
        lx, ly, lz = coords_of(last)
        col = iota_c == (i - 1).astype(jnp.float32)
        cx = jnp.where(col, lx, cx)
        cy = jnp.where(col, ly, cy)
        cz = jnp.where(col, lz, cz)
        d = (px - lx) ** 2 + (py - ly) ** 2 + (pz - lz) ** 2
        dists = jnp.minimum(dists, d)
        m = jnp.max(dists, axis=1, keepdims=True)
        nxt = jnp.min(jnp.where(dists == m, iota_n, float(_N)), axis=1,
                      keepdims=True)
        return dists, nxt, cx, cy, cz

    dists0 = jnp.full((_B, _N), jnp.inf, jnp.float32)
    last0 = jnp.zeros((_B, 1), jnp.float32)
    zc = jnp.zeros((_B, _GN), jnp.float32)
    _, last, cx, cy, cz = lax.fori_loop(1, _GN, step,
                                        (dists0, last0, zc, zc, zc))
    lx, ly, lz = coords_of(last)
    col = iota_c == float(_GN - 1)
    cx_ref[...] = jnp.where(col, lx, cx)
    cy_ref[...] = jnp.where(col, ly, cy)
    cz_ref[...] = jnp.where(col, lz, cz)


def _fps(px, py, pz):
    shp = jax.ShapeDtypeStruct((_B, _GN), jnp.float32)
    full = pl.BlockSpec((_B, _N), lambda: (0, 0))
    out = pl.BlockSpec((_B, _GN), lambda: (0, 0))
    return pl.pallas_call(
        _fps_kernel,
        grid=(),
        in_specs=[full, full, full],
        out_specs=[out, out, out],
        out_shape=[shp, shp, shp],
    )(px, py, pz)


def _ball_query(pxf, pyf, pzf, cxf, cyf, czf):
    n_workers = 32
    cpw = (_B * _GN) // n_workers
    n_chunk = _N // 16
    mesh = plsc.VectorSubcoreMesh(core_axis_name="c", subcore_axis_name="s")

    @functools.partial(
        pl.kernel,
        mesh=mesh,
        out_type=jax.ShapeDtypeStruct((_B * _GN * _GK,), jnp.int32),
        scratch_types=[
            pltpu.VMEM((_N,), jnp.float32),
            pltpu.VMEM((_N,), jnp.float32),
            pltpu.VMEM((_N,), jnp.float32),
            pltpu.VMEM((cpw,), jnp.float32),
            pltpu.VMEM((cpw,), jnp.float32),
            pltpu.VMEM((cpw,), jnp.float32),
            pltpu.VMEM((_GK + 64,), jnp.int32),
            pltpu.VMEM((cpw * _GK,), jnp.int32),
            pltpu.SemaphoreType.DMA,
        ],
        compiler_params=pltpu.CompilerParams(use_tc_tiling_on_sc=False,
                                             needs_layout_passes=False),
    )
    def k(px_hbm, py_hbm, pz_hbm, cx_hbm, cy_hbm, cz_hbm, out_hbm,
          px_v, py_v, pz_v, cx_v, cy_v, cz_v, row_v, out_v, sem):
        wid = lax.axis_index("s") * 2 + lax.axis_index("c")
        b = wid // (n_workers // _B)
        pltpu.sync_copy(px_hbm.at[pl.ds(b * _N, _N)], px_v)
        pltpu.sync_copy(py_hbm.at[pl.ds(b * _N, _N)], py_v)
        pltpu.sync_copy(pz_hbm.at[pl.ds(b * _N, _N)], pz_v)
        pltpu.sync_copy(cx_hbm.at[pl.ds(wid * cpw, cpw)], cx_v)
        pltpu.sync_copy(cy_hbm.at[pl.ds(wid * cpw, cpw)], cy_v)
        pltpu.sync_copy(cz_hbm.at[pl.ds(wid * cpw, cpw)], cz_v)
        lane = lax.broadcasted_iota(jnp.int32, (16,), 0)
        base_j = b * _N

        def per_center(s, carry):
            sidx = jnp.full((16,), s, jnp.int32)
            cxs = plsc.load_gather(cx_v, [sidx])
            cys = plsc.load_gather(cy_v, [sidx])
            czs = plsc.load_gather(cz_v, [sidx])

            def cond(c):
                i, cnt = c
                return jnp.logical_and(i < n_chunk // 4, cnt < _GK)

            def body(c):
                i, cnt = c
                for u in range(4):
                    off = i * 64 + u * 16
                    dx = px_v[pl.ds(off, 16)] - cxs
                    dy = py_v[pl.ds(off, 16)] - cys
                    dz = pz_v[pl.ds(off, 16)] - czs
                    d2 = dx * dx + dy * dy + dz * dz
                    m = d2 <= _R2
                    jv = lane + (off + base_j)
                    plsc.store_compressed(row_v.at[pl.ds(cnt, 16)], jv,
                                          mask=m)
                    cnt = cnt + jnp.max(plsc.all_reduce_population_count(m))
                return i + 1, cnt

            _, cnt = lax.while_loop(
                cond, body, (jnp.int32(0), jnp.int32(0)))
            csplat = jnp.full((16,), jnp.minimum(cnt, _GK), jnp.int32)
            v0 = row_v[pl.ds(0, 16)]
            fs = jnp.min(jnp.where(lane < csplat, v0, jnp.int32(2 ** 30)))
            first = jnp.full((16,), fs, jnp.int32)
            for h in range(_GK // 16):
                pos = lane + h * 16
                vh = row_v[pl.ds(h * 16, 16)]
                out_v[pl.ds(s * _GK + h * 16, 16)] = jnp.where(
                    pos < csplat, vh, first)
            return carry

        lax.fori_loop(0, cpw, per_center, 0)
        pltpu.sync_copy(out_v, out_hbm.at[pl.ds(wid * cpw * _GK, cpw * _GK)])

    return k(pxf, pyf, pzf, cxf, cyf, czf)


def _sc_gather(table, idx):
    rows = idx.shape[0]
    n_workers = 32
    per_w = rows // n_workers
    chunk = 512
    n_chunks = per_w // chunk
    mesh = plsc.VectorSubcoreMesh(core_axis_name="c", subcore_axis_name="s")

    @functools.partial(
        pl.kernel,
        mesh=mesh,
        out_type=jax.ShapeDtypeStruct((rows, _DIN), jnp.float32),
        scratch_types=[
            pltpu.VMEM((chunk,), jnp.int32),
            pltpu.VMEM((chunk, _DIN), jnp.float32),
            pltpu.SemaphoreType.DMA,
        ],
        compiler_params=pltpu.CompilerParams(use_tc_tiling_on_sc=False),
    )
    def k(table_hbm, idx_hbm, out_hbm, idx_v, rows_v, sem):
        wid = lax.axis_index("s") * 2 + lax.axis_index("c")
        base = wid * per_w

        def body(c, carry):
            start = base + c * chunk
            pltpu.sync_copy(idx_hbm.at[pl.ds(start, chunk)], idx_v)
            pltpu.async_copy(table_hbm.at[idx_v], rows_v, sem).wait()
            pltpu.sync_copy(rows_v, out_hbm.at[pl.ds(start, chunk)])
            return carry

        lax.fori_loop(0, n_chunks, body, 0)

    return k(table, idx)


def _acc_stats(st_ref, z):
    s = jnp.sum(z, axis=0, keepdims=True)
    q = jnp.sum(z * z, axis=0, keepdims=True)
    st = jnp.concatenate([s, q], axis=0)

    @pl.when(pl.program_id(0) == 0)
    def _():
        st_ref[...] = st

    @pl.when(pl.program_id(0) != 0)
    def _():
        st_ref[...] += st


def _affine(st, g, b):
    mu = st[0:1, :] * _INV_M
    var = st[1:2, :] * _INV_M - mu * mu
    inv = lax.rsqrt(var + _EPS)
    scale = g * inv
    shift = b - mu * scale
    return scale, shift


def _p1_kernel(g_ref, cx_ref, cy_ref, cz_ref, w0_ref, z0_ref, st_ref):
    g = g_ref[...]
    lane = lax.broadcasted_iota(jnp.int32, (_GB, _DIN), 1)
    sub = jnp.where(lane == _DF, cx_ref[...],
                    jnp.where(lane == _DF + 1, cy_ref[...],
                              jnp.where(lane == _DF + 2, cz_ref[...], 0.0)))
    x = g - sub[:, None, :]
    x2 = x.reshape(_GB * _GK, _DIN)
    z0 = jnp.dot(x2, w0_ref[...], preferred_element_type=jnp.float32)
    z0_ref[...] = z0.reshape(_GB, _GK, _DHID)
    _acc_stats(st_ref, z0)


def _p2_kernel(z0_ref, st0_ref, g0_ref, b0_ref, w1a_ref, w2_ref,
               h_ref, st1a_ref, st2_ref):
    scale, shift = _affine(st0_ref[...], g0_ref[...], b0_ref[...])
    z0 = z0_ref[...]
    h = jnp.maximum(z0 * scale[None] + shift[None], 0.0)
    h_ref[...] = h
    h2 = h.reshape(_GB * _GK, _DHID)
    z1a = jnp.dot(h2, w1a_ref[...], preferred_element_type=jnp.float32)
    z2 = jnp.dot(h2, w2_ref[...], preferred_element_type=jnp.float32)
    _acc_stats(st1a_ref, z1a)
    _acc_stats(st2_ref, z2)


def _p3_kernel(h_ref, st1a_ref, g1a_ref, b1a_ref, w1a_ref, w1b_ref,
               z1b_ref, st1b_ref):
    scale, shift = _affine(st1a_ref[...], g1a_ref[...], b1a_ref[...])
    h2 = h_ref[...].reshape(_GB * _GK, _DHID)
    z1a = jnp.dot(h2, w1a_ref[...], preferred_element_type=jnp.float32)
    t = jnp.maximum(z1a * scale + shift, 0.0)
    z1b = jnp.dot(t, w1b_ref[...], preferred_element_type=jnp.float32)
    z1b_ref[...] = z1b.reshape(_GB, _GK, _DOUT)
    _acc_stats(st1b_ref, z1b)


def _p4_kernel(z1b_ref, h_ref, st1b_ref, st2_ref,
               g1b_ref, b1b_ref, g2_ref, b2_ref, w2_ref, out_ref):
    s1b, t1b = _affine(st1b_ref[...], g1b_ref[...], b1b_ref[...])
    s2, t2 = _affine(st2_ref[...], g2_ref[...], b2_ref[...])
    h2 = h_ref[...].reshape(_GB * _GK, _DHID)
    z2 = jnp.dot(h2, w2_ref[...], preferred_element_type=jnp.float32)
    a = z1b_ref[...] * s1b[None] + t1b[None]
    c = (z2 * s2 + t2).reshape(_GB, _GK, _DOUT)
    y = jnp.maximum(a + c, 0.0)
    out_ref[...] = jnp.max(y, axis=1)


def _mlp(gathered, cxf, cyf, czf, W0, g0, b0, W1a, g1a, b1a,
         W1b, g1b, b1b, W2, g2, b2):
    n_groups = _B * _GN
    grid = (n_groups // _GB,)
    arb = pltpu.CompilerParams(dimension_semantics=("arbitrary",))

    def blk(shape_tail):
        return pl.BlockSpec((_GB,) + shape_tail, lambda i: (i,) + (0,) * len(shape_tail))

    def full2(s):
        return pl.BlockSpec(s, lambda i: (0, 0))

    st_hid = jax.ShapeDtypeStruct((2, _DHID), jnp.float32)
    st_out = jax.ShapeDtypeStruct((2, _DOUT), jnp.float32)

    z0, st0 = pl.pallas_call(
        _p1_kernel,
        grid=grid,
        in_specs=[blk((_GK, _DIN)), blk((1,)), blk((1,)), blk((1,)),
                  full2((_DIN, _DHID))],
        out_specs=[blk((_GK, _DHID)), full2((2, _DHID))],
        out_shape=[jax.ShapeDtypeStruct((n_groups, _GK, _DHID), jnp.float32),
                   st_hid],
        compiler_params=arb,
    )(gathered, cxf, cyf, czf, W0)

    h, st1a, st2 = pl.pallas_call(
        _p2_kernel,
        grid=grid,
        in_specs=[blk((_GK, _DHID)), full2((2, _DHID)),
                  full2((1, _DHID)), full2((1, _DHID)),
                  full2((_DHID, _DOUT)), full2((_DHID, _DOUT))],
        out_specs=[blk((_GK, _DHID)),
                   full2((2, _DOUT)), full2((2, _DOUT))],
        out_shape=[jax.ShapeDtypeStruct((n_groups, _GK, _DHID), jnp.float32),
                   st_out, st_out],
        compiler_params=arb,
    )(z0, st0, g0.reshape(1, _DHID), b0.reshape(1, _DHID), W1a, W2)

    z1b, st1b = pl.pallas_call(
        _p3_kernel,
        grid=grid,
        in_specs=[blk((_GK, _DHID)), full2((2, _DOUT)),
                  full2((1, _DOUT)), full2((1, _DOUT)),
                  full2((_DHID, _DOUT)), full2((_DOUT, _DOUT))],
        out_specs=[blk((_GK, _DOUT)), full2((2, _DOUT))],
        out_shape=[jax.ShapeDtypeStruct((n_groups, _GK, _DOUT), jnp.float32),
                   st_out],
        compiler_params=arb,
    )(h, st1a, g1a.reshape(1, _DOUT), b1a.reshape(1, _DOUT), W1a, W1b)

    f_ce = pl.pallas_call(
        _p4_kernel,
        grid=grid,
        in_specs=[blk((_GK, _DOUT)), blk((_GK, _DHID)),
                  full2((2, _DOUT)), full2((2, _DOUT)),
                  full2((1, _DOUT)), full2((1, _DOUT)),
                  full2((1, _DOUT)), full2((1, _DOUT)),
                  full2((_DHID, _DOUT))],
        out_specs=blk((_DOUT,)),
        out_shape=jax.ShapeDtypeStruct((n_groups, _DOUT), jnp.float32),
        compiler_params=arb,
    )(z1b, h, st1b, st2,
      g1b.reshape(1, _DOUT), b1b.reshape(1, _DOUT),
      g2.reshape(1, _DOUT), b2.reshape(1, _DOUT), W2)

    return f_ce


def kernel(f, p, W0, g0, b0, W1a, g1a, b1a, W1b, g1b, b1b, W2, g2, b2):
    px = p[:, :, 0]
    py = p[:, :, 1]
    pz = p[:, :, 2]

    cx, cy, cz = _fps(px, py, pz)
    p_ce = jnp.stack([cx, cy, cz], axis=-1)

    cxf = cx.reshape(_B * _GN, 1)
    cyf = cy.reshape(_B * _GN, 1)
    czf = cz.reshape(_B * _GN, 1)
    gidx = _ball_query(px.reshape(-1), py.reshape(-1), pz.reshape(-1),
                       cx.reshape(-1), cy.reshape(-1), cz.reshape(-1))

    table = jnp.concatenate([f, p], axis=-1).reshape(_B * _N, _DIN)
    gathered = _sc_gather(table, gidx.reshape(-1))
    gathered = gathered.reshape(_B * _GN, _GK, _DIN)

    f_ce = _mlp(gathered, cxf, cyf, czf, W0, g0, b0,
                W1a, g1a, b1a, W1b, g1b, b1b, W2, g2, b2)
    return f_ce.reshape(_B, _GN, _DOUT), p_ce


# speedup vs baseline: 1.1064x; 1.0535x over previous
import functools

import jax
import jax.numpy as jnp
import numpy as np
from jax import lax
from jax.experimental import pallas as pl
from jax.experimental.pallas import tpu as pltpu
from jax.experimental.pallas import tpu_sc as plsc

_B, _N, _DF = 4, 8192, 29
_GN, _GK = 1024, 32
_R2 = np.float32(0.15 * 0.15)
_DIN, _DHID, _DOUT = 32, 64, 128
_M = _B * _GN * _GK
_INV_M = 1.0 / _M
_EPS = 1e-5

_CB = 128
_GB = 128


def _fps_kernel(px_ref, py_ref, pz_ref, cx_ref, cy_ref, cz_ref):
    px = px_ref[...]
    py = py_ref[...]
    pz = pz_ref[...]
    iota_n = lax.broadcasted_iota(jnp.int32, (_B, _N), 1).astype(jnp.float32)
    iota_c = lax.broadcasted_iota(jnp.int32, (_B, _GN), 1).astype(jnp.float32)

    def coords_of(last):
        onehot = iota_n == last
        lx = jnp.sum(jnp.where(onehot, px, 0.0), axis=1, keepdims=True)
        ly = jnp.sum(jnp.where(onehot, py, 0.0), axis=1, keepdims=True)
        lz = jnp.sum(jnp.where(onehot, pz, 0.0), axis=1, keepdims=True)
        return lx, ly, lz

    def step(i, carry):
        dists, last, cx, cy, cz = carry
        lx, ly, lz = coords_of(last)
        col = iota_c == (i - 1).astype(jnp.float32)
        cx = jnp.where(col, lx, cx)
        cy = jnp.where(col, ly, cy)
        cz = jnp.where(col, lz, cz)
        d = (px - lx) ** 2 + (py - ly) ** 2 + (pz - lz) ** 2
        dists = jnp.minimum(dists, d)
        m = jnp.max(dists, axis=1, keepdims=True)
        nxt = jnp.min(jnp.where(dists == m, iota_n, float(_N)), axis=1,
                      keepdims=True)
        return dists, nxt, cx, cy, cz

    dists0 = jnp.full((_B, _N), jnp.inf, jnp.float32)
    last0 = jnp.zeros((_B, 1), jnp.float32)
    zc = jnp.zeros((_B, _GN), jnp.float32)
    _, last, cx, cy, cz = lax.fori_loop(1, _GN, step,
                                        (dists0, last0, zc, zc, zc))
    lx, ly, lz = coords_of(last)
    col = iota_c == float(_GN - 1)
    cx_ref[...] = jnp.where(col, lx, cx)
    cy_ref[...] = jnp.where(col, ly, cy)
    cz_ref[...] = jnp.where(col, lz, cz)


def _fps(px, py, pz):
    shp = jax.ShapeDtypeStruct((_B, _GN), jnp.float32)
    full = pl.BlockSpec((_B, _N), lambda: (0, 0))
    out = pl.BlockSpec((_B, _GN), lambda: (0, 0))
    return pl.pallas_call(
        _fps_kernel,
        grid=(),
        in_specs=[full, full, full],
        out_specs=[out, out, out],
        out_shape=[shp, shp, shp],
    )(px, py, pz)


def _ball_query(pxf, pyf, pzf, cxf, cyf, czf):
    n_workers = 32
    cpw = (_B * _GN) // n_workers
    n_chunk = _N // 16
    mesh = plsc.VectorSubcoreMesh(core_axis_name="c", subcore_axis_name="s")

    @functools.partial(
        pl.kernel,
        mesh=mesh,
        out_type=jax.ShapeDtypeStruct((_B * _GN * _GK,), jnp.int32),
        scratch_types=[
            pltpu.VMEM((_N,), jnp.float32),
            pltpu.VMEM((_N,), jnp.float32),
            pltpu.VMEM((_N,), jnp.float32),
            pltpu.VMEM((cpw,), jnp.float32),
            pltpu.VMEM((cpw,), jnp.float32),
            pltpu.VMEM((cpw,), jnp.float32),
            pltpu.VMEM((_GK + 64,), jnp.int32),
            pltpu.VMEM((cpw * _GK,), jnp.int32),
            pltpu.SemaphoreType.DMA,
        ],
        compiler_params=pltpu.CompilerParams(use_tc_tiling_on_sc=False,
                                             needs_layout_passes=False),
    )
    def k(px_hbm, py_hbm, pz_hbm, cx_hbm, cy_hbm, cz_hbm, out_hbm,
          px_v, py_v, pz_v, cx_v, cy_v, cz_v, row_v, out_v, sem):
        wid = lax.axis_index("s") * 2 + lax.axis_index("c")
        b = wid // (n_workers // _B)
        pltpu.sync_copy(px_hbm.at[pl.ds(b * _N, _N)], px_v)
        pltpu.sync_copy(py_hbm.at[pl.ds(b * _N, _N)], py_v)
        pltpu.sync_copy(pz_hbm.at[pl.ds(b * _N, _N)], pz_v)
        pltpu.sync_copy(cx_hbm.at[pl.ds(wid * cpw, cpw)], cx_v)
        pltpu.sync_copy(cy_hbm.at[pl.ds(wid * cpw, cpw)], cy_v)
        pltpu.sync_copy(cz_hbm.at[pl.ds(wid * cpw, cpw)], cz_v)
        lane = lax.broadcasted_iota(jnp.int32, (16,), 0)
        base_j = b * _N

        def per_center(s, carry):
            sidx = jnp.full((16,), s, jnp.int32)
            cxs = plsc.load_gather(cx_v, [sidx])
            cys = plsc.load_gather(cy_v, [sidx])
            czs = plsc.load_gather(cz_v, [sidx])

            def cond(c):
                i, cnt = c
                return jnp.logical_and(i < n_chunk // 4, cnt < _GK)

            def body(c):
                i, cnt = c
                for u in range(4):
                    off = i * 64 + u * 16
                    dx = px_v[pl.ds(off, 16)] - cxs
                    dy = py_v[pl.ds(off, 16)] - cys
                    dz = pz_v[pl.ds(off, 16)] - czs
                    d2 = dx * dx + dy * dy + dz * dz
                    m = d2 <= _R2
                    jv = lane + (off + base_j)
                    plsc.store_compressed(row_v.at[pl.ds(cnt, 16)], jv,
                                          mask=m)
                    cnt = cnt + plsc.all_reduce_population_count(m)[0]
                return i + 1, cnt

            _, cnt = lax.while_loop(
                cond, body, (jnp.int32(0), jnp.int32(0)))
            csplat = jnp.full((16,), jnp.minimum(cnt, _GK), jnp.int32)
            v0 = row_v[pl.ds(0, 16)]
            fs = jnp.min(jnp.where(lane < csplat, v0, jnp.int32(2 ** 30)))
            first = jnp.full((16,), fs, jnp.int32)
            for h in range(_GK // 16):
                pos = lane + h * 16
                vh = row_v[pl.ds(h * 16, 16)]
                out_v[pl.ds(s * _GK + h * 16, 16)] = jnp.where(
                    pos < csplat, vh, first)
            return carry

        lax.fori_loop(0, cpw, per_center, 0)
        pltpu.sync_copy(out_v, out_hbm.at[pl.ds(wid * cpw * _GK, cpw * _GK)])

    return k(pxf, pyf, pzf, cxf, cyf, czf)


def _sc_gather(table, idx):
    rows = idx.shape[0]
    n_workers = 32
    per_w = rows // n_workers
    chunk = 512
    n_chunks = per_w // chunk
    mesh = plsc.VectorSubcoreMesh(core_axis_name="c", subcore_axis_name="s")

    @functools.partial(
        pl.kernel,
        mesh=mesh,
        out_type=jax.ShapeDtypeStruct((rows, _DIN), jnp.float32),
        scratch_types=[
            pltpu.VMEM((chunk,), jnp.int32),
            pltpu.VMEM((chunk, _DIN), jnp.float32),
            pltpu.SemaphoreType.DMA,
        ],
        compiler_params=pltpu.CompilerParams(use_tc_tiling_on_sc=False),
    )
    def k(table_hbm, idx_hbm, out_hbm, idx_v, rows_v, sem):
        wid = lax.axis_index("s") * 2 + lax.axis_index("c")
        base = wid * per_w

        def body(c, carry):
            start = base + c * chunk
            pltpu.sync_copy(idx_hbm.at[pl.ds(start, chunk)], idx_v)
            pltpu.async_copy(table_hbm.at[idx_v], rows_v, sem).wait()
            pltpu.sync_copy(rows_v, out_hbm.at[pl.ds(start, chunk)])
            return carry

        lax.fori_loop(0, n_chunks, body, 0)

    return k(table, idx)


def _acc_stats(st_ref, z):
    s = jnp.sum(z, axis=0, keepdims=True)
    q = jnp.sum(z * z, axis=0, keepdims=True)
    st = jnp.concatenate([s, q], axis=0)

    @pl.when(pl.program_id(0) == 0)
    def _():
        st_ref[...] = st

    @pl.when(pl.program_id(0) != 0)
    def _():
        st_ref[...] += st


def _affine(st, g, b):
    mu = st[0:1, :] * _INV_M
    var = st[1:2, :] * _INV_M - mu * mu
    inv = lax.rsqrt(var + _EPS)
    scale = g * inv
    shift = b - mu * scale
    return scale, shift


def _p1_kernel(g_ref, cx_ref, cy_ref, cz_ref, w0_ref, z0_ref, st_ref):
    g = g_ref[...]
    lane = lax.broadcasted_iota(jnp.int32, (_GB, _DIN), 1)
    sub = jnp.where(lane == _DF, cx_ref[...],
                    jnp.where(lane == _DF + 1, cy_ref[...],
                              jnp.where(lane == _DF + 2, cz_ref[...], 0.0)))
    x = g - sub[:, None, :]
    x2 = x.reshape(_GB * _GK, _DIN)
    z0 = jnp.dot(x2, w0_ref[...], preferred_element_type=jnp.float32)
    z0_ref[...] = z0.reshape(_GB, _GK, _DHID)
    _acc_stats(st_ref, z0)


def _p2_kernel(z0_ref, st0_ref, g0_ref, b0_ref, w1a_ref, w2_ref,
               h_ref, st1a_ref, st2_ref):
    scale, shift = _affine(st0_ref[...], g0_ref[...], b0_ref[...])
    z0 = z0_ref[...]
    h = jnp.maximum(z0 * scale[None] + shift[None], 0.0)
    h_ref[...] = h
    h2 = h.reshape(_GB * _GK, _DHID)
    z1a = jnp.dot(h2, w1a_ref[...], preferred_element_type=jnp.float32)
    z2 = jnp.dot(h2, w2_ref[...], preferred_element_type=jnp.float32)
    _acc_stats(st1a_ref, z1a)
    _acc_stats(st2_ref, z2)


def _p3_kernel(h_ref, st1a_ref, g1a_ref, b1a_ref, w1a_ref, w1b_ref,
               z1b_ref, st1b_ref):
    scale, shift = _affine(st1a_ref[...], g1a_ref[...], b1a_ref[...])
    h2 = h_ref[...].reshape(_GB * _GK, _DHID)
    z1a = jnp.dot(h2, w1a_ref[...], preferred_element_type=jnp.float32)
    t = jnp.maximum(z1a * scale + shift, 0.0)
    z1b = jnp.dot(t, w1b_ref[...], preferred_element_type=jnp.float32)
    z1b_ref[...] = z1b.reshape(_GB, _GK, _DOUT)
    _acc_stats(st1b_ref, z1b)


def _p4_kernel(z1b_ref, h_ref, st1b_ref, st2_ref,
               g1b_ref, b1b_ref, g2_ref, b2_ref, w2_ref, out_ref):
    s1b, t1b = _affine(st1b_ref[...], g1b_ref[...], b1b_ref[...])
    s2, t2 = _affine(st2_ref[...], g2_ref[...], b2_ref[...])
    h2 = h_ref[...].reshape(_GB * _GK, _DHID)
    z2 = jnp.dot(h2, w2_ref[...], preferred_element_type=jnp.float32)
    a = z1b_ref[...] * s1b[None] + t1b[None]
    c = (z2 * s2 + t2).reshape(_GB, _GK, _DOUT)
    y = jnp.maximum(a + c, 0.0)
    out_ref[...] = jnp.max(y, axis=1)


def _mlp(gathered, cxf, cyf, czf, W0, g0, b0, W1a, g1a, b1a,
         W1b, g1b, b1b, W2, g2, b2):
    n_groups = _B * _GN
    grid = (n_groups // _GB,)
    arb = pltpu.CompilerParams(dimension_semantics=("arbitrary",))

    def blk(shape_tail):
        return pl.BlockSpec((_GB,) + shape_tail, lambda i: (i,) + (0,) * len(shape_tail))

    def full2(s):
        return pl.BlockSpec(s, lambda i: (0, 0))

    st_hid = jax.ShapeDtypeStruct((2, _DHID), jnp.float32)
    st_out = jax.ShapeDtypeStruct((2, _DOUT), jnp.float32)

    z0, st0 = pl.pallas_call(
        _p1_kernel,
        grid=grid,
        in_specs=[blk((_GK, _DIN)), blk((1,)), blk((1,)), blk((1,)),
                  full2((_DIN, _DHID))],
        out_specs=[blk((_GK, _DHID)), full2((2, _DHID))],
        out_shape=[jax.ShapeDtypeStruct((n_groups, _GK, _DHID), jnp.float32),
                   st_hid],
        compiler_params=arb,
    )(gathered, cxf, cyf, czf, W0)

    h, st1a, st2 = pl.pallas_call(
        _p2_kernel,
        grid=grid,
        in_specs=[blk((_GK, _DHID)), full2((2, _DHID)),
                  full2((1, _DHID)), full2((1, _DHID)),
                  full2((_DHID, _DOUT)), full2((_DHID, _DOUT))],
        out_specs=[blk((_GK, _DHID)),
                   full2((2, _DOUT)), full2((2, _DOUT))],
        out_shape=[jax.ShapeDtypeStruct((n_groups, _GK, _DHID), jnp.float32),
                   st_out, st_out],
        compiler_params=arb,
    )(z0, st0, g0.reshape(1, _DHID), b0.reshape(1, _DHID), W1a, W2)

    z1b, st1b = pl.pallas_call(
        _p3_kernel,
        grid=grid,
        in_specs=[blk((_GK, _DHID)), full2((2, _DOUT)),
                  full2((1, _DOUT)), full2((1, _DOUT)),
                  full2((_DHID, _DOUT)), full2((_DOUT, _DOUT))],
        out_specs=[blk((_GK, _DOUT)), full2((2, _DOUT))],
        out_shape=[jax.ShapeDtypeStruct((n_groups, _GK, _DOUT), jnp.float32),
                   st_out],
        compiler_params=arb,
    )(h, st1a, g1a.reshape(1, _DOUT), b1a.reshape(1, _DOUT), W1a, W1b)

    f_ce = pl.pallas_call(
        _p4_kernel,
        grid=grid,
        in_specs=[blk((_GK, _DOUT)), blk((_GK, _DHID)),
                  full2((2, _DOUT)), full2((2, _DOUT)),
                  full2((1, _DOUT)), full2((1, _DOUT)),
                  full2((1, _DOUT)), full2((1, _DOUT)),
                  full2((_DHID, _DOUT))],
        out_specs=blk((_DOUT,)),
        out_shape=jax.ShapeDtypeStruct((n_groups, _DOUT), jnp.float32),
        compiler_params=arb,
    )(z1b, h, st1b, st2,
      g1b.reshape(1, _DOUT), b1b.reshape(1, _DOUT),
      g2.reshape(1, _DOUT), b2.reshape(1, _DOUT), W2)

    return f_ce


def kernel(f, p, W0, g0, b0, W1a, g1a, b1a, W1b, g1b, b1b, W2, g2, b2):
    px = p[:, :, 0]
    py = p[:, :, 1]
    pz = p[:, :, 2]

    cx, cy, cz = _fps(px, py, pz)
    p_ce = jnp.stack([cx, cy, cz], axis=-1)

    cxf = cx.reshape(_B * _GN, 1)
    cyf = cy.reshape(_B * _GN, 1)
    czf = cz.reshape(_B * _GN, 1)
    gidx = _ball_query(px.reshape(-1), py.reshape(-1), pz.reshape(-1),
                       cx.reshape(-1), cy.reshape(-1), cz.reshape(-1))

    table = jnp.concatenate([f, p], axis=-1).reshape(_B * _N, _DIN)
    gathered = _sc_gather(table, gidx.reshape(-1))
    gathered = gathered.reshape(_B * _GN, _GK, _DIN)

    f_ce = _mlp(gathered, cxf, cyf, czf, W0, g0, b0,
                W1a, g1a, b1a, W1b, g1b, b1b, W2, g2, b2)
    return f_ce.reshape(_B, _GN, _DOUT), p_ce


# speedup vs baseline: 1.1450x; 1.0349x over previous
import functools

import jax
import jax.numpy as jnp
import numpy as np
from jax import lax
from jax.experimental import pallas as pl
from jax.experimental.pallas import tpu as pltpu
from jax.experimental.pallas import tpu_sc as plsc

_B, _N, _DF = 4, 8192, 29
_GN, _GK = 1024, 32
_R2 = np.float32(0.15 * 0.15)
_DIN, _DHID, _DOUT = 32, 64, 128
_M = _B * _GN * _GK
_INV_M = 1.0 / _M
_EPS = 1e-5

_CB = 128
_GB = 128


def _fps_kernel(px_ref, py_ref, pz_ref, cx_ref, cy_ref, cz_ref):
    px = px_ref[...]
    py = py_ref[...]
    pz = pz_ref[...]
    iota_n = lax.broadcasted_iota(jnp.int32, (_B, _N), 1).astype(jnp.float32)
    iota_c = lax.broadcasted_iota(jnp.int32, (_B, _GN), 1).astype(jnp.float32)

    def coords_of(last):
        onehot = iota_n == last
        lx = jnp.sum(jnp.where(onehot, px, 0.0), axis=1, keepdims=True)
        ly = jnp.sum(jnp.where(onehot, py, 0.0), axis=1, keepdims=True)
        lz = jnp.sum(jnp.where(onehot, pz, 0.0), axis=1, keepdims=True)
        return lx, ly, lz

    def step(i, carry):
        dists, last, cx, cy, cz = carry
        lx, ly, lz = coords_of(last)
        col = iota_c == (i - 1).astype(jnp.float32)
        cx = jnp.where(col, lx, cx)
        cy = jnp.where(col, ly, cy)
        cz = jnp.where(col, lz, cz)
        d = (px - lx) ** 2 + (py - ly) ** 2 + (pz - lz) ** 2
        dists = jnp.minimum(dists, d)
        m = jnp.max(dists, axis=1, keepdims=True)
        nxt = jnp.min(jnp.where(dists == m, iota_n, float(_N)), axis=1,
                      keepdims=True)
        return dists, nxt, cx, cy, cz

    dists0 = jnp.full((_B, _N), jnp.inf, jnp.float32)
    last0 = jnp.zeros((_B, 1), jnp.float32)
    zc = jnp.zeros((_B, _GN), jnp.float32)
    _, last, cx, cy, cz = lax.fori_loop(1, _GN, step,
                                        (dists0, last0, zc, zc, zc))
    lx, ly, lz = coords_of(last)
    col = iota_c == float(_GN - 1)
    cx_ref[...] = jnp.where(col, lx, cx)
    cy_ref[...] = jnp.where(col, ly, cy)
    cz_ref[...] = jnp.where(col, lz, cz)


def _fps(px, py, pz):
    shp = jax.ShapeDtypeStruct((_B, _GN), jnp.float32)
    full = pl.BlockSpec((_B, _N), lambda: (0, 0))
    out = pl.BlockSpec((_B, _GN), lambda: (0, 0))
    return pl.pallas_call(
        _fps_kernel,
        grid=(),
        in_specs=[full, full, full],
        out_specs=[out, out, out],
        out_shape=[shp, shp, shp],
    )(px, py, pz)


def _ball_query(pxf, pyf, pzf, cxf, cyf, czf):
    n_workers = 32
    cpw = (_B * _GN) // n_workers
    n_chunk = _N // 16
    mesh = plsc.VectorSubcoreMesh(core_axis_name="c", subcore_axis_name="s")

    @functools.partial(
        pl.kernel,
        mesh=mesh,
        out_type=jax.ShapeDtypeStruct((_B * _GN * _GK,), jnp.int32),
        scratch_types=[
            pltpu.VMEM((_N,), jnp.float32),
            pltpu.VMEM((_N,), jnp.float32),
            pltpu.VMEM((_N,), jnp.float32),
            pltpu.VMEM((cpw,), jnp.float32),
            pltpu.VMEM((cpw,), jnp.float32),
            pltpu.VMEM((cpw,), jnp.float32),
            pltpu.VMEM((_GK + 64,), jnp.int32),
            pltpu.VMEM((cpw * _GK,), jnp.int32),
            pltpu.SemaphoreType.DMA,
        ],
        compiler_params=pltpu.CompilerParams(use_tc_tiling_on_sc=False,
                                             needs_layout_passes=False),
    )
    def k(px_hbm, py_hbm, pz_hbm, cx_hbm, cy_hbm, cz_hbm, out_hbm,
          px_v, py_v, pz_v, cx_v, cy_v, cz_v, row_v, out_v, sem):
        wid = lax.axis_index("s") * 2 + lax.axis_index("c")
        b = wid // (n_workers // _B)
        pltpu.sync_copy(px_hbm.at[pl.ds(b * _N, _N)], px_v)
        pltpu.sync_copy(py_hbm.at[pl.ds(b * _N, _N)], py_v)
        pltpu.sync_copy(pz_hbm.at[pl.ds(b * _N, _N)], pz_v)
        pltpu.sync_copy(cx_hbm.at[pl.ds(wid * cpw, cpw)], cx_v)
        pltpu.sync_copy(cy_hbm.at[pl.ds(wid * cpw, cpw)], cy_v)
        pltpu.sync_copy(cz_hbm.at[pl.ds(wid * cpw, cpw)], cz_v)
        lane = lax.broadcasted_iota(jnp.int32, (16,), 0)
        base_j = b * _N

        def per_center(s, carry):
            sidx = jnp.full((16,), s, jnp.int32)
            cxs = plsc.load_gather(cx_v, [sidx])
            cys = plsc.load_gather(cy_v, [sidx])
            czs = plsc.load_gather(cz_v, [sidx])

            def cond(c):
                i, cnt = c
                return jnp.logical_and(i < n_chunk // 4, cnt < _GK)

            def body(c):
                i, cnt = c
                for u in range(4):
                    off = i * 64 + u * 16
                    dx = px_v[pl.ds(off, 16)] - cxs
                    dy = py_v[pl.ds(off, 16)] - cys
                    dz = pz_v[pl.ds(off, 16)] - czs
                    d2 = dx * dx + dy * dy + dz * dz
                    m = d2 <= _R2
                    jv = lane + (off + base_j)
                    plsc.store_compressed(row_v.at[pl.ds(cnt, 16)], jv,
                                          mask=m)
                    cnt = cnt + plsc.all_reduce_population_count(m)[0]
                return i + 1, cnt

            _, cnt = lax.while_loop(
                cond, body, (jnp.int32(0), jnp.int32(0)))
            csplat = jnp.full((16,), jnp.minimum(cnt, _GK), jnp.int32)
            v0 = row_v[pl.ds(0, 16)]
            fs = jnp.min(jnp.where(lane < csplat, v0, jnp.int32(2 ** 30)))
            first = jnp.full((16,), fs, jnp.int32)
            for h in range(_GK // 16):
                pos = lane + h * 16
                vh = row_v[pl.ds(h * 16, 16)]
                out_v[pl.ds(s * _GK + h * 16, 16)] = jnp.where(
                    pos < csplat, vh, first)
            return carry

        lax.fori_loop(0, cpw, per_center, 0)
        pltpu.sync_copy(out_v, out_hbm.at[pl.ds(wid * cpw * _GK, cpw * _GK)])

    return k(pxf, pyf, pzf, cxf, cyf, czf)


def _sc_gather(table, idx):
    rows = idx.shape[0]
    n_workers = 32
    per_w = rows // n_workers
    chunk = 512
    n_chunks = per_w // chunk
    mesh = plsc.VectorSubcoreMesh(core_axis_name="c", subcore_axis_name="s")

    @functools.partial(
        pl.kernel,
        mesh=mesh,
        out_type=jax.ShapeDtypeStruct((rows, _DIN), jnp.float32),
        scratch_types=[
            pltpu.VMEM((chunk,), jnp.int32),
            pltpu.VMEM((chunk, _DIN), jnp.float32),
            pltpu.SemaphoreType.DMA,
        ],
        compiler_params=pltpu.CompilerParams(use_tc_tiling_on_sc=False),
    )
    def k(table_hbm, idx_hbm, out_hbm, idx_v, rows_v, sem):
        wid = lax.axis_index("s") * 2 + lax.axis_index("c")
        base = wid * per_w

        def body(c, carry):
            start = base + c * chunk
            pltpu.sync_copy(idx_hbm.at[pl.ds(start, chunk)], idx_v)
            pltpu.async_copy(table_hbm.at[idx_v], rows_v, sem).wait()
            pltpu.sync_copy(rows_v, out_hbm.at[pl.ds(start, chunk)])
            return carry

        lax.fori_loop(0, n_chunks, body, 0)

    return k(table, idx)


def _acc_stats(st_ref, z):
    s = jnp.sum(z, axis=0, keepdims=True)
    q = jnp.sum(z * z, axis=0, keepdims=True)
    st = jnp.concatenate([s, q], axis=0)

    @pl.when(pl.program_id(0) == 0)
    def _():
        st_ref[...] = st

    @pl.when(pl.program_id(0) != 0)
    def _():
        st_ref[...] += st


def _affine(st, g, b):
    mu = st[0:1, :] * _INV_M
    var = st[1:2, :] * _INV_M - mu * mu
    inv = lax.rsqrt(var + _EPS)
    scale = g * inv
    shift = b - mu * scale
    return scale, shift


def _p1_kernel(g_ref, cx_ref, cy_ref, cz_ref, w0_ref, z0_ref, st_ref):
    g = g_ref[...]
    lane = lax.broadcasted_iota(jnp.int32, (_GB, _DIN), 1)
    sub = jnp.where(lane == _DF, cx_ref[...],
                    jnp.where(lane == _DF + 1, cy_ref[...],
                              jnp.where(lane == _DF + 2, cz_ref[...], 0.0)))
    x = g - sub[:, None, :]
    x2 = x.reshape(_GB * _GK, _DIN)
    z0 = jnp.dot(x2, w0_ref[...], preferred_element_type=jnp.float32)
    z0_ref[...] = z0.reshape(_GB, _GK, _DHID)
    _acc_stats(st_ref, z0)


def _p2_kernel(z0_ref, st0_ref, g0_ref, b0_ref, w1a_ref, w2_ref,
               h_ref, st1a_ref, st2_ref):
    scale, shift = _affine(st0_ref[...], g0_ref[...], b0_ref[...])
    z0 = z0_ref[...]
    h = jnp.maximum(z0 * scale[None] + shift[None], 0.0)
    h_ref[...] = h
    h2 = h.reshape(_GB * _GK, _DHID)
    z1a = jnp.dot(h2, w1a_ref[...], preferred_element_type=jnp.float32)
    z2 = jnp.dot(h2, w2_ref[...], preferred_element_type=jnp.float32)
    _acc_stats(st1a_ref, z1a)
    _acc_stats(st2_ref, z2)


def _p3_kernel(h_ref, st1a_ref, g1a_ref, b1a_ref, w1a_ref, w1b_ref,
               z1b_ref, st1b_ref):
    scale, shift = _affine(st1a_ref[...], g1a_ref[...], b1a_ref[...])
    h2 = h_ref[...].reshape(_GB * _GK, _DHID)
    z1a = jnp.dot(h2, w1a_ref[...], preferred_element_type=jnp.float32)
    t = jnp.maximum(z1a * scale + shift, 0.0)
    z1b = jnp.dot(t, w1b_ref[...], preferred_element_type=jnp.float32)
    z1b_ref[...] = z1b.reshape(_GB, _GK, _DOUT)
    _acc_stats(st1b_ref, z1b)


def _p4_kernel(z1b_ref, h_ref, st1b_ref, st2_ref,
               g1b_ref, b1b_ref, g2_ref, b2_ref, w2_ref, out_ref):
    s1b, t1b = _affine(st1b_ref[...], g1b_ref[...], b1b_ref[...])
    s2, t2 = _affine(st2_ref[...], g2_ref[...], b2_ref[...])
    h2 = h_ref[...].reshape(_GB * _GK, _DHID)
    z2 = jnp.dot(h2, w2_ref[...], preferred_element_type=jnp.float32)
    a = z1b_ref[...] * s1b[None] + t1b[None]
    c = (z2 * s2 + t2).reshape(_GB, _GK, _DOUT)
    y = jnp.maximum(a + c, 0.0)
    out_ref[...] = jnp.max(y, axis=1)


def _mlp(gathered, cxf, cyf, czf, W0, g0, b0, W1a, g1a, b1a,
         W1b, g1b, b1b, W2, g2, b2):
    n_groups = _B * _GN
    grid = (n_groups // _GB,)
    arb = pltpu.CompilerParams(dimension_semantics=("arbitrary",))

    def blk(shape_tail):
        return pl.BlockSpec((_GB,) + shape_tail, lambda i: (i,) + (0,) * len(shape_tail))

    def full2(s):
        return pl.BlockSpec(s, lambda i: (0, 0))

    st_hid = jax.ShapeDtypeStruct((2, _DHID), jnp.float32)
    st_out = jax.ShapeDtypeStruct((2, _DOUT), jnp.float32)

    z0, st0 = pl.pallas_call(
        _p1_kernel,
        grid=grid,
        in_specs=[blk((_GK, _DIN)), blk((1,)), blk((1,)), blk((1,)),
                  full2((_DIN, _DHID))],
        out_specs=[blk((_GK, _DHID)), full2((2, _DHID))],
        out_shape=[jax.ShapeDtypeStruct((n_groups, _GK, _DHID), jnp.float32),
                   st_hid],
        compiler_params=arb,
    )(gathered, cxf, cyf, czf, W0)

    h, st1a, st2 = pl.pallas_call(
        _p2_kernel,
        grid=grid,
        in_specs=[blk((_GK, _DHID)), full2((2, _DHID)),
                  full2((1, _DHID)), full2((1, _DHID)),
                  full2((_DHID, _DOUT)), full2((_DHID, _DOUT))],
        out_specs=[blk((_GK, _DHID)),
                   full2((2, _DOUT)), full2((2, _DOUT))],
        out_shape=[jax.ShapeDtypeStruct((n_groups, _GK, _DHID), jnp.float32),
                   st_out, st_out],
        compiler_params=arb,
    )(z0, st0, g0.reshape(1, _DHID), b0.reshape(1, _DHID), W1a, W2)

    z1b, st1b = pl.pallas_call(
        _p3_kernel,
        grid=grid,
        in_specs=[blk((_GK, _DHID)), full2((2, _DOUT)),
                  full2((1, _DOUT)), full2((1, _DOUT)),
                  full2((_DHID, _DOUT)), full2((_DOUT, _DOUT))],
        out_specs=[blk((_GK, _DOUT)), full2((2, _DOUT))],
        out_shape=[jax.ShapeDtypeStruct((n_groups, _GK, _DOUT), jnp.float32),
                   st_out],
        compiler_params=arb,
    )(h, st1a, g1a.reshape(1, _DOUT), b1a.reshape(1, _DOUT), W1a, W1b)

    f_ce = pl.pallas_call(
        _p4_kernel,
        grid=grid,
        in_specs=[blk((_GK, _DOUT)), blk((_GK, _DHID)),
                  full2((2, _DOUT)), full2((2, _DOUT)),
                  full2((1, _DOUT)), full2((1, _DOUT)),
                  full2((1, _DOUT)), full2((1, _DOUT)),
                  full2((_DHID, _DOUT))],
        out_specs=blk((_DOUT,)),
        out_shape=jax.ShapeDtypeStruct((n_groups, _DOUT), jnp.float32),
        compiler_params=arb,
    )(z1b, h, st1b, st2,
      g1b.reshape(1, _DOUT), b1b.reshape(1, _DOUT),
      g2.reshape(1, _DOUT), b2.reshape(1, _DOUT), W2)

    return f_ce


def kernel(f, p, W0, g0, b0, W1a, g1a, b1a, W1b, g1b, b1b, W2, g2, b2):
    px = p[:, :, 0]
    py = p[:, :, 1]
    pz = p[:, :, 2]

    cx, cy, cz = _fps(px, py, pz)
    p_ce = jnp.stack([cx, cy, cz], axis=-1)

    cxf = cx.reshape(_B * _GN, 1)
    cyf = cy.reshape(_B * _GN, 1)
    czf = cz.reshape(_B * _GN, 1)

    def perm(a):
        return a.reshape(_B, _GN // 8, 8).transpose(0, 2, 1).reshape(-1)

    gidx = _ball_query(px.reshape(-1), py.reshape(-1), pz.reshape(-1),
                       perm(cx), perm(cy), perm(cz))
    gidx = (gidx.reshape(_B, 8, _GN // 8, _GK)
            .transpose(0, 2, 1, 3).reshape(-1))

    table = jnp.concatenate([f, p], axis=-1).reshape(_B * _N, _DIN)
    gathered = _sc_gather(table, gidx.reshape(-1))
    gathered = gathered.reshape(_B * _GN, _GK, _DIN)

    f_ce = _mlp(gathered, cxf, cyf, czf, W0, g0, b0,
                W1a, g1a, b1a, W1b, g1b, b1b, W2, g2, b2)
    return f_ce.reshape(_B, _GN, _DOUT), p_ce


# speedup vs baseline: 1.1780x; 1.0288x over previous
import functools

import jax
import jax.numpy as jnp
import numpy as np
from jax import lax
from jax.experimental import pallas as pl
from jax.experimental.pallas import tpu as pltpu
from jax.experimental.pallas import tpu_sc as plsc

_B, _N, _DF = 4, 8192, 29
_GN, _GK = 1024, 32
_R2 = np.float32(0.15 * 0.15)
_DIN, _DHID, _DOUT = 32, 64, 128
_M = _B * _GN * _GK
_INV_M = 1.0 / _M
_EPS = 1e-5

_CB = 128
_GB = 256


def _fps_kernel(px_ref, py_ref, pz_ref, cx_ref, cy_ref, cz_ref):
    px = px_ref[...]
    py = py_ref[...]
    pz = pz_ref[...]
    iota_n = lax.broadcasted_iota(jnp.int32, (_B, _N), 1).astype(jnp.float32)
    iota_c = lax.broadcasted_iota(jnp.int32, (_B, _GN), 1).astype(jnp.float32)

    def coords_of(last):
        onehot = iota_n == last
        lx = jnp.sum(jnp.where(onehot, px, 0.0), axis=1, keepdims=True)
        ly = jnp.sum(jnp.where(onehot, py, 0.0), axis=1, keepdims=True)
        lz = jnp.sum(jnp.where(onehot, pz, 0.0), axis=1, keepdims=True)
        return lx, ly, lz

    def step(i, carry):
        dists, last, cx, cy, cz = carry
        lx, ly, lz = coords_of(last)
        col = iota_c == (i - 1).astype(jnp.float32)
        cx = jnp.where(col, lx, cx)
        cy = jnp.where(col, ly, cy)
        cz = jnp.where(col, lz, cz)
        d = (px - lx) ** 2 + (py - ly) ** 2 + (pz - lz) ** 2
        dists = jnp.minimum(dists, d)
        m = jnp.max(dists, axis=1, keepdims=True)
        nxt = jnp.min(jnp.where(dists == m, iota_n, float(_N)), axis=1,
                      keepdims=True)
        return dists, nxt, cx, cy, cz

    dists0 = jnp.full((_B, _N), jnp.inf, jnp.float32)
    last0 = jnp.zeros((_B, 1), jnp.float32)
    zc = jnp.zeros((_B, _GN), jnp.float32)
    _, last, cx, cy, cz = lax.fori_loop(1, _GN, step,
                                        (dists0, last0, zc, zc, zc))
    lx, ly, lz = coords_of(last)
    col = iota_c == float(_GN - 1)
    cx_ref[...] = jnp.where(col, lx, cx)
    cy_ref[...] = jnp.where(col, ly, cy)
    cz_ref[...] = jnp.where(col, lz, cz)


def _fps(px, py, pz):
    shp = jax.ShapeDtypeStruct((_B, _GN), jnp.float32)
    full = pl.BlockSpec((_B, _N), lambda: (0, 0))
    out = pl.BlockSpec((_B, _GN), lambda: (0, 0))
    return pl.pallas_call(
        _fps_kernel,
        grid=(),
        in_specs=[full, full, full],
        out_specs=[out, out, out],
        out_shape=[shp, shp, shp],
    )(px, py, pz)


def _ball_query(pxf, pyf, pzf, cxf, cyf, czf):
    n_workers = 32
    cpw = (_B * _GN) // n_workers
    n_chunk = _N // 16
    mesh = plsc.VectorSubcoreMesh(core_axis_name="c", subcore_axis_name="s")

    @functools.partial(
        pl.kernel,
        mesh=mesh,
        out_type=jax.ShapeDtypeStruct((_B * _GN * _GK,), jnp.int32),
        scratch_types=[
            pltpu.VMEM((_N,), jnp.float32),
            pltpu.VMEM((_N,), jnp.float32),
            pltpu.VMEM((_N,), jnp.float32),
            pltpu.VMEM((cpw,), jnp.float32),
            pltpu.VMEM((cpw,), jnp.float32),
            pltpu.VMEM((cpw,), jnp.float32),
            pltpu.VMEM((_GK + 64,), jnp.int32),
            pltpu.VMEM((cpw * _GK,), jnp.int32),
            pltpu.SemaphoreType.DMA,
        ],
        compiler_params=pltpu.CompilerParams(use_tc_tiling_on_sc=False,
                                             needs_layout_passes=False),
    )
    def k(px_hbm, py_hbm, pz_hbm, cx_hbm, cy_hbm, cz_hbm, out_hbm,
          px_v, py_v, pz_v, cx_v, cy_v, cz_v, row_v, out_v, sem):
        wid = lax.axis_index("s") * 2 + lax.axis_index("c")
        b = wid // (n_workers // _B)
        pltpu.sync_copy(px_hbm.at[pl.ds(b * _N, _N)], px_v)
        pltpu.sync_copy(py_hbm.at[pl.ds(b * _N, _N)], py_v)
        pltpu.sync_copy(pz_hbm.at[pl.ds(b * _N, _N)], pz_v)
        pltpu.sync_copy(cx_hbm.at[pl.ds(wid * cpw, cpw)], cx_v)
        pltpu.sync_copy(cy_hbm.at[pl.ds(wid * cpw, cpw)], cy_v)
        pltpu.sync_copy(cz_hbm.at[pl.ds(wid * cpw, cpw)], cz_v)
        lane = lax.broadcasted_iota(jnp.int32, (16,), 0)
        base_j = b * _N

        def per_center(s, carry):
            sidx = jnp.full((16,), s, jnp.int32)
            cxs = plsc.load_gather(cx_v, [sidx])
            cys = plsc.load_gather(cy_v, [sidx])
            czs = plsc.load_gather(cz_v, [sidx])

            def cond(c):
                i, cnt = c
                return jnp.logical_and(i < n_chunk // 4, cnt < _GK)

            def body(c):
                i, cnt = c
                for u in range(4):
                    off = i * 64 + u * 16
                    dx = px_v[pl.ds(off, 16)] - cxs
                    dy = py_v[pl.ds(off, 16)] - cys
                    dz = pz_v[pl.ds(off, 16)] - czs
                    d2 = dx * dx + dy * dy + dz * dz
                    m = d2 <= _R2
                    jv = lane + (off + base_j)
                    plsc.store_compressed(row_v.at[pl.ds(cnt, 16)], jv,
                                          mask=m)
                    cnt = cnt + plsc.all_reduce_population_count(m)[0]
                return i + 1, cnt

            _, cnt = lax.while_loop(
                cond, body, (jnp.int32(0), jnp.int32(0)))
            csplat = jnp.full((16,), jnp.minimum(cnt, _GK), jnp.int32)
            v0 = row_v[pl.ds(0, 16)]
            fs = jnp.min(jnp.where(lane < csplat, v0, jnp.int32(2 ** 30)))
            first = jnp.full((16,), fs, jnp.int32)
            for h in range(_GK // 16):
                pos = lane + h * 16
                vh = row_v[pl.ds(h * 16, 16)]
                out_v[pl.ds(s * _GK + h * 16, 16)] = jnp.where(
                    pos < csplat, vh, first)
            return carry

        lax.fori_loop(0, cpw, per_center, 0)
        pltpu.sync_copy(out_v, out_hbm.at[pl.ds(wid * cpw * _GK, cpw * _GK)])

    return k(pxf, pyf, pzf, cxf, cyf, czf)


def _sc_gather(table, idx):
    rows = idx.shape[0]
    n_workers = 32
    per_w = rows // n_workers
    chunk = 512
    n_chunks = per_w // chunk
    mesh = plsc.VectorSubcoreMesh(core_axis_name="c", subcore_axis_name="s")

    @functools.partial(
        pl.kernel,
        mesh=mesh,
        out_type=jax.ShapeDtypeStruct((rows, _DIN), jnp.float32),
        scratch_types=[
            pltpu.VMEM((chunk,), jnp.int32),
            pltpu.VMEM((chunk, _DIN), jnp.float32),
            pltpu.SemaphoreType.DMA,
        ],
        compiler_params=pltpu.CompilerParams(use_tc_tiling_on_sc=False),
    )
    def k(table_hbm, idx_hbm, out_hbm, idx_v, rows_v, sem):
        wid = lax.axis_index("s") * 2 + lax.axis_index("c")
        base = wid * per_w

        def body(c, carry):
            start = base + c * chunk
            pltpu.sync_copy(idx_hbm.at[pl.ds(start, chunk)], idx_v)
            pltpu.async_copy(table_hbm.at[idx_v], rows_v, sem).wait()
            pltpu.sync_copy(rows_v, out_hbm.at[pl.ds(start, chunk)])
            return carry

        lax.fori_loop(0, n_chunks, body, 0)

    return k(table, idx)


def _acc_stats(st_ref, z):
    s = jnp.sum(z, axis=0, keepdims=True)
    q = jnp.sum(z * z, axis=0, keepdims=True)
    st = jnp.concatenate([s, q], axis=0)

    @pl.when(pl.program_id(0) == 0)
    def _():
        st_ref[...] = st

    @pl.when(pl.program_id(0) != 0)
    def _():
        st_ref[...] += st


def _affine(st, g, b):
    mu = st[0:1, :] * _INV_M
    var = st[1:2, :] * _INV_M - mu * mu
    inv = lax.rsqrt(var + _EPS)
    scale = g * inv
    shift = b - mu * scale
    return scale, shift


def _p1_kernel(g_ref, cx_ref, cy_ref, cz_ref, w0_ref, z0_ref, st_ref):
    g = g_ref[...]
    lane = lax.broadcasted_iota(jnp.int32, (_GB, _DIN), 1)
    sub = jnp.where(lane == _DF, cx_ref[...],
                    jnp.where(lane == _DF + 1, cy_ref[...],
                              jnp.where(lane == _DF + 2, cz_ref[...], 0.0)))
    x = g - sub[:, None, :]
    x2 = x.reshape(_GB * _GK, _DIN)
    z0 = jnp.dot(x2, w0_ref[...], preferred_element_type=jnp.float32)
    z0_ref[...] = z0.reshape(_GB, _GK, _DHID)
    _acc_stats(st_ref, z0)


def _p2_kernel(z0_ref, st0_ref, g0_ref, b0_ref, w1a_ref, w2_ref,
               h_ref, st1a_ref, st2_ref):
    scale, shift = _affine(st0_ref[...], g0_ref[...], b0_ref[...])
    z0 = z0_ref[...]
    h = jnp.maximum(z0 * scale[None] + shift[None], 0.0)
    h_ref[...] = h
    h2 = h.reshape(_GB * _GK, _DHID)
    z1a = jnp.dot(h2, w1a_ref[...], preferred_element_type=jnp.float32)
    z2 = jnp.dot(h2, w2_ref[...], preferred_element_type=jnp.float32)
    _acc_stats(st1a_ref, z1a)
    _acc_stats(st2_ref, z2)


def _p3_kernel(h_ref, st1a_ref, g1a_ref, b1a_ref, w1a_ref, w1b_ref,
               z1b_ref, st1b_ref):
    scale, shift = _affine(st1a_ref[...], g1a_ref[...], b1a_ref[...])
    h2 = h_ref[...].reshape(_GB * _GK, _DHID)
    z1a = jnp.dot(h2, w1a_ref[...], preferred_element_type=jnp.float32)
    t = jnp.maximum(z1a * scale + shift, 0.0)
    z1b = jnp.dot(t, w1b_ref[...], preferred_element_type=jnp.float32)
    z1b_ref[...] = z1b.reshape(_GB, _GK, _DOUT)
    _acc_stats(st1b_ref, z1b)


def _p4_kernel(z1b_ref, h_ref, st1b_ref, st2_ref,
               g1b_ref, b1b_ref, g2_ref, b2_ref, w2_ref, out_ref):
    s1b, t1b = _affine(st1b_ref[...], g1b_ref[...], b1b_ref[...])
    s2, t2 = _affine(st2_ref[...], g2_ref[...], b2_ref[...])
    h2 = h_ref[...].reshape(_GB * _GK, _DHID)
    z2 = jnp.dot(h2, w2_ref[...], preferred_element_type=jnp.float32)
    a = z1b_ref[...] * s1b[None] + t1b[None]
    c = (z2 * s2 + t2).reshape(_GB, _GK, _DOUT)
    y = jnp.maximum(a + c, 0.0)
    out_ref[...] = jnp.max(y, axis=1)


def _mlp(gathered, cxf, cyf, czf, W0, g0, b0, W1a, g1a, b1a,
         W1b, g1b, b1b, W2, g2, b2):
    n_groups = _B * _GN
    grid = (n_groups // _GB,)
    arb = pltpu.CompilerParams(dimension_semantics=("arbitrary",))

    def blk(shape_tail):
        return pl.BlockSpec((_GB,) + shape_tail, lambda i: (i,) + (0,) * len(shape_tail))

    def full2(s):
        return pl.BlockSpec(s, lambda i: (0, 0))

    st_hid = jax.ShapeDtypeStruct((2, _DHID), jnp.float32)
    st_out = jax.ShapeDtypeStruct((2, _DOUT), jnp.float32)

    z0, st0 = pl.pallas_call(
        _p1_kernel,
        grid=grid,
        in_specs=[blk((_GK, _DIN)), blk((1,)), blk((1,)), blk((1,)),
                  full2((_DIN, _DHID))],
        out_specs=[blk((_GK, _DHID)), full2((2, _DHID))],
        out_shape=[jax.ShapeDtypeStruct((n_groups, _GK, _DHID), jnp.float32),
                   st_hid],
        compiler_params=arb,
    )(gathered, cxf, cyf, czf, W0)

    h, st1a, st2 = pl.pallas_call(
        _p2_kernel,
        grid=grid,
        in_specs=[blk((_GK, _DHID)), full2((2, _DHID)),
                  full2((1, _DHID)), full2((1, _DHID)),
                  full2((_DHID, _DOUT)), full2((_DHID, _DOUT))],
        out_specs=[blk((_GK, _DHID)),
                   full2((2, _DOUT)), full2((2, _DOUT))],
        out_shape=[jax.ShapeDtypeStruct((n_groups, _GK, _DHID), jnp.float32),
                   st_out, st_out],
        compiler_params=arb,
    )(z0, st0, g0.reshape(1, _DHID), b0.reshape(1, _DHID), W1a, W2)

    z1b, st1b = pl.pallas_call(
        _p3_kernel,
        grid=grid,
        in_specs=[blk((_GK, _DHID)), full2((2, _DOUT)),
                  full2((1, _DOUT)), full2((1, _DOUT)),
                  full2((_DHID, _DOUT)), full2((_DOUT, _DOUT))],
        out_specs=[blk((_GK, _DOUT)), full2((2, _DOUT))],
        out_shape=[jax.ShapeDtypeStruct((n_groups, _GK, _DOUT), jnp.float32),
                   st_out],
        compiler_params=arb,
    )(h, st1a, g1a.reshape(1, _DOUT), b1a.reshape(1, _DOUT), W1a, W1b)

    f_ce = pl.pallas_call(
        _p4_kernel,
        grid=grid,
        in_specs=[blk((_GK, _DOUT)), blk((_GK, _DHID)),
                  full2((2, _DOUT)), full2((2, _DOUT)),
                  full2((1, _DOUT)), full2((1, _DOUT)),
                  full2((1, _DOUT)), full2((1, _DOUT)),
                  full2((_DHID, _DOUT))],
        out_specs=blk((_DOUT,)),
        out_shape=jax.ShapeDtypeStruct((n_groups, _DOUT), jnp.float32),
        compiler_params=arb,
    )(z1b, h, st1b, st2,
      g1b.reshape(1, _DOUT), b1b.reshape(1, _DOUT),
      g2.reshape(1, _DOUT), b2.reshape(1, _DOUT), W2)

    return f_ce


def kernel(f, p, W0, g0, b0, W1a, g1a, b1a, W1b, g1b, b1b, W2, g2, b2):
    px = p[:, :, 0]
    py = p[:, :, 1]
    pz = p[:, :, 2]

    cx, cy, cz = _fps(px, py, pz)
    p_ce = jnp.stack([cx, cy, cz], axis=-1)

    cxf = cx.reshape(_B * _GN, 1)
    cyf = cy.reshape(_B * _GN, 1)
    czf = cz.reshape(_B * _GN, 1)

    def perm(a):
        return a.reshape(_B, _GN // 8, 8).transpose(0, 2, 1).reshape(-1)

    gidx = _ball_query(px.reshape(-1), py.reshape(-1), pz.reshape(-1),
                       perm(cx), perm(cy), perm(cz))
    gidx = (gidx.reshape(_B, 8, _GN // 8, _GK)
            .transpose(0, 2, 1, 3).reshape(-1))

    table = jnp.concatenate([f, p], axis=-1).reshape(_B * _N, _DIN)
    gathered = _sc_gather(table, gidx.reshape(-1))
    gathered = gathered.reshape(_B * _GN, _GK, _DIN)

    f_ce = _mlp(gathered, cxf, cyf, czf, W0, g0, b0,
                W1a, g1a, b1a, W1b, g1b, b1b, W2, g2, b2)
    return f_ce.reshape(_B, _GN, _DOUT), p_ce
